# Initial kernel scaffold; baseline (speedup 1.0000x reference)
import functools

import jax
import jax.numpy as jnp
from jax import lax
from jax.experimental import pallas as pl
from jax.experimental.pallas import tpu as pltpu
from jax.experimental.pallas import tpu_sc as plsc

N = 10000
E = 320000
H = 128
G = 64
L = 5
F = 3

NC = 2
NS = 16
NW = NC * NS
EPW = E // NW
CH = 80
NCH = EPW // CH
RPT = N // NS

BN = 2000
BE = 4000


def _mesh():
    return plsc.VectorSubcoreMesh(core_axis_name="c", subcore_axis_name="s")


def _d2_body(posT, srcr, dstr, out, px, py, pz, si, di, ov):
    c = lax.axis_index("c")
    s = lax.axis_index("s")
    wid = s * NC + c
    base = wid * EPW
    pltpu.sync_copy(posT.at[0], px)
    pltpu.sync_copy(posT.at[1], py)
    pltpu.sync_copy(posT.at[2], pz)
    pltpu.sync_copy(srcr.at[pl.ds(base, EPW)], si)
    pltpu.sync_copy(dstr.at[pl.ds(base, EPW)], di)

    def body(i, carry):
        isrc = si[pl.ds(i * 16, 16)]
        idst = di[pl.ds(i * 16, 16)]
        dx = plsc.load_gather(px, [isrc]) - plsc.load_gather(px, [idst])
        dy = plsc.load_gather(py, [isrc]) - plsc.load_gather(py, [idst])
        dz = plsc.load_gather(pz, [isrc]) - plsc.load_gather(pz, [idst])
        ov[pl.ds(i * 16, 16)] = dx * dx + dy * dy + dz * dz
        return carry

    lax.fori_loop(0, EPW // 16, body, 0)
    pltpu.sync_copy(ov, out.at[pl.ds(base, EPW)])


def _d2_call(posT, src, dst):
    fn = pl.kernel(
        _d2_body,
        out_type=jax.ShapeDtypeStruct((E,), jnp.float32),
        mesh=_mesh(),
        scratch_types=[
            pltpu.VMEM((N,), jnp.float32),
            pltpu.VMEM((N,), jnp.float32),
            pltpu.VMEM((N,), jnp.float32),
            pltpu.VMEM((EPW,), jnp.int32),
            pltpu.VMEM((EPW,), jnp.int32),
            pltpu.VMEM((EPW,), jnp.float32),
        ],
    )
    return fn(posT, src, dst)


def _gather_body(d_hbm, s_hbm, dst3, src3, c_hbm, di, si, gd, gs, sem):
    c = lax.axis_index("c")
    s = lax.axis_index("s")
    wid = s * NC + c
    base = wid * EPW
    pltpu.sync_copy(dst3.at[wid], di)
    pltpu.sync_copy(src3.at[wid], si)

    def body(j, carry):
        cp1 = pltpu.async_copy(d_hbm.at[di.at[j]], gd, sem)
        cp2 = pltpu.async_copy(s_hbm.at[si.at[j]], gs, sem)
        cp1.wait()
        cp2.wait()

        def addrow(r, carry2):
            for k in range(16):
                sl = pl.ds(k * 16, 16)
                gd[r, sl] = gd[r, sl] + gs[r, sl]
            return carry2

        lax.fori_loop(0, CH, addrow, 0)
        pltpu.sync_copy(gd, c_hbm.at[pl.ds(base + j * CH, CH)])
        return carry

    lax.fori_loop(0, NCH, body, 0)


def _gather_call(d_tab, s_tab, dst3, src3):
    fn = pl.kernel(
        _gather_body,
        out_type=jax.ShapeDtypeStruct((E, 2 * H), jnp.float32),
        mesh=_mesh(),
        scratch_types=[
            pltpu.VMEM((NCH, CH), jnp.int32),
            pltpu.VMEM((NCH, CH), jnp.int32),
            pltpu.VMEM((CH, 2 * H), jnp.float32),
            pltpu.VMEM((CH, 2 * H), jnp.float32),
            pltpu.SemaphoreType.DMA,
        ],
    )
    return fn(d_tab, s_tab, dst3, src3)


def _scatter_body(m_hbm, dst3, zeros_hbm, out, di, mv, agg):
    c = lax.axis_index("c")
    s = lax.axis_index("s")
    wid = s * NC + c
    pltpu.sync_copy(zeros_hbm.at[pl.ds(s * RPT, RPT)], agg.at[pl.ds(s * RPT, RPT)])
    pltpu.sync_copy(dst3.at[wid], di)
    plsc.subcore_barrier()

    def body(j, carry):
        pltpu.sync_copy(m_hbm.at[pl.ds(wid * EPW + j * CH, CH)], mv)
        pltpu.sync_copy(mv, agg.at[di.at[j]], add=True)
        return carry

    lax.fori_loop(0, NCH, body, 0)
    plsc.subcore_barrier()
    pltpu.sync_copy(agg.at[pl.ds(s * RPT, RPT)], out.at[c].at[pl.ds(s * RPT, RPT)])


def _scatter_call(m, dst3, zeros_n):
    fn = pl.kernel(
        _scatter_body,
        out_type=jax.ShapeDtypeStruct((NC, N, H), jnp.float32),
        mesh=_mesh(),
        scratch_types=[
            pltpu.VMEM((NCH, CH), jnp.int32),
            pltpu.VMEM((CH, H), jnp.float32),
            pltpu.VMEM_SHARED((N, H), jnp.float32),
        ],
    )
    return fn(m, dst3, zeros_n)


def _embed_body(a_ref, emb_ref, o_ref):
    oh = (a_ref[...] == lax.broadcasted_iota(jnp.int32, (1, 128), 1)).astype(jnp.float32)
    o_ref[...] = jnp.dot(oh, emb_ref[...], preferred_element_type=jnp.float32)


def _embed_call(atoms2, embp):
    return pl.pallas_call(
        _embed_body,
        grid=(N // BN,),
        in_specs=[
            pl.BlockSpec((BN, 1), lambda i: (i, 0)),
            pl.BlockSpec((128, H), lambda i: (0, 0)),
        ],
        out_specs=pl.BlockSpec((BN, H), lambda i: (i, 0)),
        out_shape=jax.ShapeDtypeStruct((N, H), jnp.float32),
    )(atoms2, embp)


def _proj_body(x_ref, p0_ref, p1_ref, wd_ref, bd_ref, ws_ref, xn_ref, d_ref, s_ref):
    xn = x_ref[...] + p0_ref[...] + p1_ref[...]
    xn_ref[...] = xn
    d_ref[...] = jnp.dot(xn, wd_ref[...], preferred_element_type=jnp.float32) + bd_ref[...]
    s_ref[...] = jnp.dot(xn, ws_ref[...], preferred_element_type=jnp.float32)


def _proj_call(x, p0, p1, wd, bd, ws):
    return pl.pallas_call(
        _proj_body,
        grid=(N // BN,),
        in_specs=[
            pl.BlockSpec((BN, H), lambda i: (i, 0)),
            pl.BlockSpec((BN, H), lambda i: (i, 0)),
            pl.BlockSpec((BN, H), lambda i: (i, 0)),
            pl.BlockSpec((H, 2 * H), lambda i: (0, 0)),
            pl.BlockSpec((1, 2 * H), lambda i: (0, 0)),
            pl.BlockSpec((H, 2 * H), lambda i: (0, 0)),
        ],
        out_specs=[
            pl.BlockSpec((BN, H), lambda i: (i, 0)),
            pl.BlockSpec((BN, 2 * H), lambda i: (i, 0)),
            pl.BlockSpec((BN, 2 * H), lambda i: (i, 0)),
        ],
        out_shape=[
            jax.ShapeDtypeStruct((N, H), jnp.float32),
            jax.ShapeDtypeStruct((N, 2 * H), jnp.float32),
            jax.ShapeDtypeStruct((N, 2 * H), jnp.float32),
        ],
    )(x, p0, p1, wd, bd, ws)


def _act_body(c_ref, d2_ref, wf2_ref, ws2_ref, m_ref):
    cb = c_ref[...]
    e = jnp.sqrt(d2_ref[...])
    u = cb[:, :H] + e * wf2_ref[...]
    v = cb[:, H:] + e * ws2_ref[...]
    gate = 1.0 / (1.0 + jnp.exp(-u))
    core = jnp.maximum(v, 0.0) + jnp.log(1.0 + jnp.exp(-jnp.abs(v)))
    m_ref[...] = gate * core


def _act_call(c_arr, d2c, wf2, ws2):
    return pl.pallas_call(
        _act_body,
        grid=(E // BE,),
        in_specs=[
            pl.BlockSpec((BE, 2 * H), lambda i: (i, 0)),
            pl.BlockSpec((BE, 1), lambda i: (i, 0)),
            pl.BlockSpec((1, H), lambda i: (0, 0)),
            pl.BlockSpec((1, H), lambda i: (0, 0)),
        ],
        out_specs=pl.BlockSpec((BE, H), lambda i: (i, 0)),
        out_shape=jax.ShapeDtypeStruct((E, H), jnp.float32),
    )(c_arr, d2c, wf2, ws2)


def _pool_body(x_ref, p0_ref, p1_ref, b_ref, fcw_ref, fcb_ref, ow_ref, ob_ref,
               o_ref, acc, cnt):
    i = pl.program_id(0)

    @pl.when(i == 0)
    def _():
        acc[...] = jnp.zeros_like(acc)
        cnt[...] = jnp.zeros_like(cnt)

    xb = x_ref[...] + p0_ref[...] + p1_ref[...]
    oh = (b_ref[...] == lax.broadcasted_iota(jnp.int32, (1, G), 1)).astype(jnp.float32)
    acc[...] += lax.dot_general(oh, xb, (((0,), (0,)), ((), ())),
                                preferred_element_type=jnp.float32)
    cnt[...] += lax.dot_general(oh, jnp.ones((BN, 1), jnp.float32),
                                (((0,), (0,)), ((), ())),
                                preferred_element_type=jnp.float32)

    @pl.when(i == pl.num_programs(0) - 1)
    def _():
        h = acc[...] / jnp.maximum(cnt[...], 1.0)
        for l in range(F):
            h = jnp.dot(h, fcw_ref[l], preferred_element_type=jnp.float32) + fcb_ref[l]
        o_ref[...] = jnp.dot(h, ow_ref[...], preferred_element_type=jnp.float32) + ob_ref[...]


def _pool_call(x, p0, p1, batch2, fcW, fcb3, outW, outb2):
    return pl.pallas_call(
        _pool_body,
        grid=(N // BN,),
        in_specs=[
            pl.BlockSpec((BN, H), lambda i: (i, 0)),
            pl.BlockSpec((BN, H), lambda i: (i, 0)),
            pl.BlockSpec((BN, H), lambda i: (i, 0)),
            pl.BlockSpec((BN, 1), lambda i: (i, 0)),
            pl.BlockSpec((F, H, H), lambda i: (0, 0, 0)),
            pl.BlockSpec((F, 1, H), lambda i: (0, 0, 0)),
            pl.BlockSpec((H, 1), lambda i: (0, 0)),
            pl.BlockSpec((1, 1), lambda i: (0, 0)),
        ],
        out_specs=pl.BlockSpec((G, 1), lambda i: (0, 0)),
        out_shape=jax.ShapeDtypeStruct((G, 1), jnp.float32),
        scratch_shapes=[
            pltpu.VMEM((G, H), jnp.float32),
            pltpu.VMEM((G, 1), jnp.float32),
        ],
    )(x, p0, p1, batch2, fcW, fcb3, outW, outb2)


def kernel(atoms, pos, edge_index, batch, emb, Wf, bf, Ws, bs, fcW, fcb, outW, outb):
    atoms2 = atoms.astype(jnp.int32).reshape(N, 1)
    src = edge_index[0].astype(jnp.int32)
    dst = edge_index[1].astype(jnp.int32)
    batch2 = batch.astype(jnp.int32).reshape(N, 1)
    posT = pos.T
    embp = jnp.pad(emb, ((0, 128 - emb.shape[0]), (0, 0)))

    WD = jnp.concatenate([Wf[:, :H, :], Ws[:, :H, :]], axis=2)
    WSrc = jnp.concatenate([Wf[:, H:2 * H, :], Ws[:, H:2 * H, :]], axis=2)
    bD = jnp.concatenate([bf, bs], axis=1).reshape(L, 1, 2 * H)
    wf2 = Wf[:, 2 * H, :].reshape(L, 1, H)
    ws2 = Ws[:, 2 * H, :].reshape(L, 1, H)

    dst3 = dst.reshape(NW, NCH, CH)
    src3 = src.reshape(NW, NCH, CH)
    zeros_n = jnp.zeros((N, H), jnp.float32)

    d2 = _d2_call(posT, src, dst)
    d2c = d2.reshape(E, 1)

    x = _embed_call(atoms2, embp)
    p0 = zeros_n
    p1 = zeros_n
    for l in range(L):
        x, d_tab, s_tab = _proj_call(x, p0, p1, WD[l], bD[l], WSrc[l])
        c_arr = _gather_call(d_tab, s_tab, dst3, src3)
        m = _act_call(c_arr, d2c, wf2[l], ws2[l])
        parts = _scatter_call(m, dst3, zeros_n)
        p0 = parts[0]
        p1 = parts[1]

    return _pool_call(x, p0, p1, batch2, fcW, fcb.reshape(F, 1, H), outW,
                      outb.reshape(1, 1))


# speedup vs baseline: 2.5741x; 2.5741x over previous
import functools

import jax
import jax.numpy as jnp
from jax import lax
from jax.experimental import pallas as pl
from jax.experimental.pallas import tpu as pltpu
from jax.experimental.pallas import tpu_sc as plsc

N = 10000
E = 320000
H = 128
G = 64
L = 5
F = 3

NC = 2
NS = 16
NW = NC * NS
EPW = E // NW
CH = 80
NCH = EPW // CH
RPT = 624
RREM = N - NS * RPT

BN = 2000
BE = 4000


def _mesh():
    return plsc.VectorSubcoreMesh(core_axis_name="c", subcore_axis_name="s")


def _diff_body(posx, posy, posz, srcr, dstr, ox, oy, oz, px, py, pz, si, di,
               vx, vy, vz):
    c = lax.axis_index("c")
    s = lax.axis_index("s")
    wid = s * NC + c
    base = wid * EPW
    pltpu.sync_copy(posx, px)
    pltpu.sync_copy(posy, py)
    pltpu.sync_copy(posz, pz)
    pltpu.sync_copy(srcr.at[pl.ds(base, EPW)], si)
    pltpu.sync_copy(dstr.at[pl.ds(base, EPW)], di)

    def body(i, carry):
        sl = pl.ds(i * 16, 16)
        isrc = si[sl]
        idst = di[sl]
        vx[sl] = plsc.load_gather(px, [isrc]) - plsc.load_gather(px, [idst])
        vy[sl] = plsc.load_gather(py, [isrc]) - plsc.load_gather(py, [idst])
        vz[sl] = plsc.load_gather(pz, [isrc]) - plsc.load_gather(pz, [idst])
        return carry

    lax.fori_loop(0, EPW // 16, body, 0)
    pltpu.sync_copy(vx, ox.at[pl.ds(base, EPW)])
    pltpu.sync_copy(vy, oy.at[pl.ds(base, EPW)])
    pltpu.sync_copy(vz, oz.at[pl.ds(base, EPW)])


def _diff_call(posx, posy, posz, src, dst):
    fn = pl.kernel(
        _diff_body,
        out_type=[jax.ShapeDtypeStruct((E,), jnp.float32)] * 3,
        mesh=_mesh(),
        compiler_params=pltpu.CompilerParams(needs_layout_passes=False),
        scratch_types=[
            pltpu.VMEM((N,), jnp.float32),
            pltpu.VMEM((N,), jnp.float32),
            pltpu.VMEM((N,), jnp.float32),
            pltpu.VMEM((EPW,), jnp.int32),
            pltpu.VMEM((EPW,), jnp.int32),
            pltpu.VMEM((EPW,), jnp.float32),
            pltpu.VMEM((EPW,), jnp.float32),
            pltpu.VMEM((EPW,), jnp.float32),
        ],
    )
    return fn(posx, posy, posz, src, dst)


def _gather_body(d_hbm, s_hbm, dst3, src3, c_hbm, di, si, gd, gs, sem):
    c = lax.axis_index("c")
    s = lax.axis_index("s")
    wid = s * NC + c
    base = wid * EPW
    pltpu.sync_copy(dst3.at[wid], di)
    pltpu.sync_copy(src3.at[wid], si)

    def body(j, carry):
        cp1 = pltpu.async_copy(d_hbm.at[di.at[j]], gd, sem)
        cp2 = pltpu.async_copy(s_hbm.at[si.at[j]], gs, sem)
        cp1.wait()
        cp2.wait()

        def addrow(r, carry2):
            for k in range(16):
                sl = pl.ds(k * 16, 16)
                gd[r, sl] = gd[r, sl] + gs[r, sl]
            return carry2

        lax.fori_loop(0, CH, addrow, 0)
        pltpu.sync_copy(gd, c_hbm.at[pl.ds(base + j * CH, CH)])
        return carry

    lax.fori_loop(0, NCH, body, 0)


def _gather_call(d_tab, s_tab, dst3, src3):
    fn = pl.kernel(
        _gather_body,
        out_type=jax.ShapeDtypeStruct((E, 2 * H), jnp.float32),
        mesh=_mesh(),
        compiler_params=pltpu.CompilerParams(needs_layout_passes=False),
        scratch_types=[
            pltpu.VMEM((NCH, CH), jnp.int32),
            pltpu.VMEM((NCH, CH), jnp.int32),
            pltpu.VMEM((CH, 2 * H), jnp.float32),
            pltpu.VMEM((CH, 2 * H), jnp.float32),
            pltpu.SemaphoreType.DMA,
        ],
    )
    return fn(d_tab, s_tab, dst3, src3)


def _scatter_body(m_hbm, dst3, zeros_hbm, out, di, mv, agg):
    c = lax.axis_index("c")
    s = lax.axis_index("s")
    wid = s * NC + c
    pltpu.sync_copy(zeros_hbm.at[pl.ds(s * RPT, RPT)], agg.at[pl.ds(s * RPT, RPT)])

    @pl.when(s == 0)
    def _():
        pltpu.sync_copy(zeros_hbm.at[pl.ds(NS * RPT, RREM)],
                        agg.at[pl.ds(NS * RPT, RREM)])

    pltpu.sync_copy(dst3.at[wid], di)
    plsc.subcore_barrier()

    def body(j, carry):
        pltpu.sync_copy(m_hbm.at[pl.ds(wid * EPW + j * CH, CH)], mv)
        pltpu.sync_copy(mv, agg.at[di.at[j]], add=True)
        return carry

    lax.fori_loop(0, NCH, body, 0)
    plsc.subcore_barrier()
    pltpu.sync_copy(agg.at[pl.ds(s * RPT, RPT)], out.at[c].at[pl.ds(s * RPT, RPT)])

    @pl.when(s == 0)
    def _():
        pltpu.sync_copy(agg.at[pl.ds(NS * RPT, RREM)],
                        out.at[c].at[pl.ds(NS * RPT, RREM)])


def _scatter_call(m, dst3, zeros_n):
    fn = pl.kernel(
        _scatter_body,
        out_type=jax.ShapeDtypeStruct((NC, N, H), jnp.float32),
        mesh=_mesh(),
        compiler_params=pltpu.CompilerParams(needs_layout_passes=False),
        scratch_types=[
            pltpu.VMEM((NCH, CH), jnp.int32),
            pltpu.VMEM((CH, H), jnp.float32),
            pltpu.VMEM_SHARED((N, H), jnp.float32),
        ],
    )
    return fn(m, dst3, zeros_n)


def _embed_body(a_ref, emb_ref, o_ref):
    oh = (a_ref[...] == lax.broadcasted_iota(jnp.int32, (1, 128), 1)).astype(jnp.float32)
    o_ref[...] = jnp.dot(oh, emb_ref[...], preferred_element_type=jnp.float32, precision=lax.Precision.HIGHEST)


def _embed_call(atoms2, embp):
    return pl.pallas_call(
        _embed_body,
        grid=(N // BN,),
        in_specs=[
            pl.BlockSpec((BN, 1), lambda i: (i, 0)),
            pl.BlockSpec((128, H), lambda i: (0, 0)),
        ],
        out_specs=pl.BlockSpec((BN, H), lambda i: (i, 0)),
        out_shape=jax.ShapeDtypeStruct((N, H), jnp.float32),
    )(atoms2, embp)


def _proj_body(x_ref, p0_ref, p1_ref, wd_ref, bd_ref, ws_ref, xn_ref, d_ref, s_ref):
    xn = x_ref[...] + p0_ref[...] + p1_ref[...]
    xn_ref[...] = xn
    d_ref[...] = jnp.dot(xn, wd_ref[...], preferred_element_type=jnp.float32) + bd_ref[...]
    s_ref[...] = jnp.dot(xn, ws_ref[...], preferred_element_type=jnp.float32)


def _proj_call(x, p0, p1, wd, bd, ws):
    return pl.pallas_call(
        _proj_body,
        grid=(N // BN,),
        in_specs=[
            pl.BlockSpec((BN, H), lambda i: (i, 0)),
            pl.BlockSpec((BN, H), lambda i: (i, 0)),
            pl.BlockSpec((BN, H), lambda i: (i, 0)),
            pl.BlockSpec((H, 2 * H), lambda i: (0, 0)),
            pl.BlockSpec((1, 2 * H), lambda i: (0, 0)),
            pl.BlockSpec((H, 2 * H), lambda i: (0, 0)),
        ],
        out_specs=[
            pl.BlockSpec((BN, H), lambda i: (i, 0)),
            pl.BlockSpec((BN, 2 * H), lambda i: (i, 0)),
            pl.BlockSpec((BN, 2 * H), lambda i: (i, 0)),
        ],
        out_shape=[
            jax.ShapeDtypeStruct((N, H), jnp.float32),
            jax.ShapeDtypeStruct((N, 2 * H), jnp.float32),
            jax.ShapeDtypeStruct((N, 2 * H), jnp.float32),
        ],
    )(x, p0, p1, wd, bd, ws)


def _act_body(c_ref, dx_ref, dy_ref, dz_ref, wf2_ref, ws2_ref, m_ref):
    cb = c_ref[...]
    dx = dx_ref[...]
    dy = dy_ref[...]
    dz = dz_ref[...]
    e = jnp.sqrt(dx * dx + dy * dy + dz * dz)
    u = cb[:, :H] + e * wf2_ref[...]
    v = cb[:, H:] + e * ws2_ref[...]
    gate = 1.0 / (1.0 + jnp.exp(-u))
    core = jnp.maximum(v, 0.0) + jnp.log(1.0 + jnp.exp(-jnp.abs(v)))
    m_ref[...] = gate * core


def _act_call(c_arr, dxc, dyc, dzc, wf2, ws2):
    return pl.pallas_call(
        _act_body,
        grid=(E // BE,),
        in_specs=[
            pl.BlockSpec((BE, 2 * H), lambda i: (i, 0)),
            pl.BlockSpec((BE, 1), lambda i: (i, 0)),
            pl.BlockSpec((BE, 1), lambda i: (i, 0)),
            pl.BlockSpec((BE, 1), lambda i: (i, 0)),
            pl.BlockSpec((1, H), lambda i: (0, 0)),
            pl.BlockSpec((1, H), lambda i: (0, 0)),
        ],
        out_specs=pl.BlockSpec((BE, H), lambda i: (i, 0)),
        out_shape=jax.ShapeDtypeStruct((E, H), jnp.float32),
    )(c_arr, dxc, dyc, dzc, wf2, ws2)


def _pool_body(x_ref, p0_ref, p1_ref, b_ref, fcw_ref, fcb_ref, ow_ref, ob_ref,
               o_ref, acc, cnt):
    i = pl.program_id(0)

    @pl.when(i == 0)
    def _():
        acc[...] = jnp.zeros_like(acc)
        cnt[...] = jnp.zeros_like(cnt)

    xb = x_ref[...] + p0_ref[...] + p1_ref[...]
    oh = (b_ref[...] == lax.broadcasted_iota(jnp.int32, (1, G), 1)).astype(jnp.float32)
    acc[...] += lax.dot_general(oh, xb, (((0,), (0,)), ((), ())),
                                preferred_element_type=jnp.float32, precision=lax.Precision.HIGHEST)
    cnt[...] += lax.dot_general(oh, jnp.ones((BN, 1), jnp.float32),
                                (((0,), (0,)), ((), ())),
                                preferred_element_type=jnp.float32, precision=lax.Precision.HIGHEST)

    @pl.when(i == pl.num_programs(0) - 1)
    def _():
        h = acc[...] / jnp.maximum(cnt[...], 1.0)
        for l in range(F):
            h = jnp.dot(h, fcw_ref[l], preferred_element_type=jnp.float32) + fcb_ref[l]
        o_ref[...] = jnp.dot(h, ow_ref[...], preferred_element_type=jnp.float32) + ob_ref[...]


def _pool_call(x, p0, p1, batch2, fcW, fcb3, outW, outb2):
    return pl.pallas_call(
        _pool_body,
        grid=(N // BN,),
        in_specs=[
            pl.BlockSpec((BN, H), lambda i: (i, 0)),
            pl.BlockSpec((BN, H), lambda i: (i, 0)),
            pl.BlockSpec((BN, H), lambda i: (i, 0)),
            pl.BlockSpec((BN, 1), lambda i: (i, 0)),
            pl.BlockSpec((F, H, H), lambda i: (0, 0, 0)),
            pl.BlockSpec((F, 1, H), lambda i: (0, 0, 0)),
            pl.BlockSpec((H, 1), lambda i: (0, 0)),
            pl.BlockSpec((1, 1), lambda i: (0, 0)),
        ],
        out_specs=pl.BlockSpec((G, 1), lambda i: (0, 0)),
        out_shape=jax.ShapeDtypeStruct((G, 1), jnp.float32),
        scratch_shapes=[
            pltpu.VMEM((G, H), jnp.float32),
            pltpu.VMEM((G, 1), jnp.float32),
        ],
    )(x, p0, p1, batch2, fcW, fcb3, outW, outb2)


def kernel(atoms, pos, edge_index, batch, emb, Wf, bf, Ws, bs, fcW, fcb, outW, outb):
    atoms2 = atoms.astype(jnp.int32).reshape(N, 1)
    src = edge_index[0].astype(jnp.int32)
    dst = edge_index[1].astype(jnp.int32)
    batch2 = batch.astype(jnp.int32).reshape(N, 1)
    posx = pos[:, 0]
    posy = pos[:, 1]
    posz = pos[:, 2]
    embp = jnp.pad(emb, ((0, 128 - emb.shape[0]), (0, 0)))

    WD = jnp.concatenate([Wf[:, :H, :], Ws[:, :H, :]], axis=2)
    WSrc = jnp.concatenate([Wf[:, H:2 * H, :], Ws[:, H:2 * H, :]], axis=2)
    bD = jnp.concatenate([bf, bs], axis=1).reshape(L, 1, 2 * H)
    wf2 = Wf[:, 2 * H, :].reshape(L, 1, H)
    ws2 = Ws[:, 2 * H, :].reshape(L, 1, H)

    dst3 = dst.reshape(NW, NCH, CH)
    src3 = src.reshape(NW, NCH, CH)
    zeros_n = jnp.zeros((N, H), jnp.float32)

    dxe, dye, dze = _diff_call(posx, posy, posz, src, dst)
    dxc = dxe.reshape(E, 1)
    dyc = dye.reshape(E, 1)
    dzc = dze.reshape(E, 1)

    x = _embed_call(atoms2, embp)
    p0 = zeros_n
    p1 = zeros_n
    for l in range(L):
        x, d_tab, s_tab = _proj_call(x, p0, p1, WD[l], bD[l], WSrc[l])
        c_arr = _gather_call(d_tab, s_tab, dst3, src3)
        m = _act_call(c_arr, dxc, dyc, dzc, wf2[l], ws2[l])
        parts = _scatter_call(m, dst3, zeros_n)
        p0 = parts[0]
        p1 = parts[1]

    return _pool_call(x, p0, p1, batch2, fcW, fcb.reshape(F, 1, H), outW,
                      outb.reshape(1, 1))


# speedup vs baseline: 3.3078x; 1.2850x over previous
import functools

import jax
import jax.numpy as jnp
from jax import lax
from jax.experimental import pallas as pl
from jax.experimental.pallas import tpu as pltpu
from jax.experimental.pallas import tpu_sc as plsc

N = 10000
E = 320000
H = 128
G = 64
L = 5
F = 3

NC = 2
NS = 16
NW = NC * NS
EPW = E // NW
CH = 80
NCH = EPW // CH
RPT = 624
RREM = N - NS * RPT

BN = 2000
BE = 4000


def _mesh():
    return plsc.VectorSubcoreMesh(core_axis_name="c", subcore_axis_name="s")


def _diff_body(posx, posy, posz, srcr, dstr, ox, oy, oz, px, py, pz, si, di,
               vx, vy, vz):
    c = lax.axis_index("c")
    s = lax.axis_index("s")
    wid = s * NC + c
    base = wid * EPW
    pltpu.sync_copy(posx, px)
    pltpu.sync_copy(posy, py)
    pltpu.sync_copy(posz, pz)
    pltpu.sync_copy(srcr.at[pl.ds(base, EPW)], si)
    pltpu.sync_copy(dstr.at[pl.ds(base, EPW)], di)

    def body(i, carry):
        sl = pl.ds(i * 16, 16)
        isrc = si[sl]
        idst = di[sl]
        vx[sl] = plsc.load_gather(px, [isrc]) - plsc.load_gather(px, [idst])
        vy[sl] = plsc.load_gather(py, [isrc]) - plsc.load_gather(py, [idst])
        vz[sl] = plsc.load_gather(pz, [isrc]) - plsc.load_gather(pz, [idst])
        return carry

    lax.fori_loop(0, EPW // 16, body, 0)
    pltpu.sync_copy(vx, ox.at[pl.ds(base, EPW)])
    pltpu.sync_copy(vy, oy.at[pl.ds(base, EPW)])
    pltpu.sync_copy(vz, oz.at[pl.ds(base, EPW)])


def _diff_call(posx, posy, posz, src, dst):
    fn = pl.kernel(
        _diff_body,
        out_type=[jax.ShapeDtypeStruct((E,), jnp.float32)] * 3,
        mesh=_mesh(),
        compiler_params=pltpu.CompilerParams(needs_layout_passes=False),
        scratch_types=[
            pltpu.VMEM((N,), jnp.float32),
            pltpu.VMEM((N,), jnp.float32),
            pltpu.VMEM((N,), jnp.float32),
            pltpu.VMEM((EPW,), jnp.int32),
            pltpu.VMEM((EPW,), jnp.int32),
            pltpu.VMEM((EPW,), jnp.float32),
            pltpu.VMEM((EPW,), jnp.float32),
            pltpu.VMEM((EPW,), jnp.float32),
        ],
    )
    return fn(posx, posy, posz, src, dst)


def _add_rows(gd, gs):
    def addrow(r, carry2):
        for k in range(16):
            sl = pl.ds(k * 16, 16)
            gd[r, sl] = gd[r, sl] + gs[r, sl]
        return carry2

    lax.fori_loop(0, CH, addrow, 0)


def _gather_body(d_hbm, s_hbm, dst3, src3, c_hbm, di, si,
                 gda, gsa, gdb, gsb, semga, semgb, semsa, semsb):
    c = lax.axis_index("c")
    s = lax.axis_index("s")
    wid = s * NC + c
    base = wid * EPW
    pltpu.sync_copy(dst3.at[wid], di)
    pltpu.sync_copy(src3.at[wid], si)

    def gath(j, gd, gs, sem):
        pltpu.async_copy(d_hbm.at[di.at[j]], gd, sem)
        pltpu.async_copy(s_hbm.at[si.at[j]], gs, sem)

    def wait2(gd, gs, sem):
        pltpu.make_async_copy(d_hbm.at[di.at[0]], gd, sem).wait()
        pltpu.make_async_copy(s_hbm.at[si.at[0]], gs, sem).wait()

    gath(0, gda, gsa, semga)

    def body(j2, carry):
        ja = 2 * j2
        jb = ja + 1

        @pl.when(j2 > 0)
        def _():
            pltpu.make_async_copy(gdb, c_hbm.at[pl.ds(0, CH)], semsb).wait()

        gath(jb, gdb, gsb, semgb)
        wait2(gda, gsa, semga)
        _add_rows(gda, gsa)
        pltpu.async_copy(gda, c_hbm.at[pl.ds(base + ja * CH, CH)], semsa)
        wait2(gdb, gsb, semgb)
        _add_rows(gdb, gsb)
        pltpu.async_copy(gdb, c_hbm.at[pl.ds(base + jb * CH, CH)], semsb)
        pltpu.make_async_copy(gda, c_hbm.at[pl.ds(0, CH)], semsa).wait()
        gath(ja + 2, gda, gsa, semga)
        return carry

    lax.fori_loop(0, (NCH - 1) // 2, body, 0)

    wait2(gda, gsa, semga)
    _add_rows(gda, gsa)
    pltpu.sync_copy(gda, c_hbm.at[pl.ds(base + (NCH - 1) * CH, CH)])
    pltpu.make_async_copy(gdb, c_hbm.at[pl.ds(0, CH)], semsb).wait()


def _gather_call(d_tab, s_tab, dst3, src3):
    fn = pl.kernel(
        _gather_body,
        out_type=jax.ShapeDtypeStruct((E, 2 * H), jnp.float32),
        mesh=_mesh(),
        compiler_params=pltpu.CompilerParams(needs_layout_passes=False),
        scratch_types=[
            pltpu.VMEM((NCH, CH), jnp.int32),
            pltpu.VMEM((NCH, CH), jnp.int32),
            pltpu.VMEM((CH, 2 * H), jnp.float32),
            pltpu.VMEM((CH, 2 * H), jnp.float32),
            pltpu.VMEM((CH, 2 * H), jnp.float32),
            pltpu.VMEM((CH, 2 * H), jnp.float32),
            pltpu.SemaphoreType.DMA,
            pltpu.SemaphoreType.DMA,
            pltpu.SemaphoreType.DMA,
            pltpu.SemaphoreType.DMA,
        ],
    )
    return fn(d_tab, s_tab, dst3, src3)


def _scatter_body(m_hbm, dst3, zeros_hbm, out, di, mva, mvb, agg, semma, semmb):
    c = lax.axis_index("c")
    s = lax.axis_index("s")
    wid = s * NC + c
    pltpu.sync_copy(zeros_hbm.at[pl.ds(s * RPT, RPT)], agg.at[pl.ds(s * RPT, RPT)])

    @pl.when(s == 0)
    def _():
        pltpu.sync_copy(zeros_hbm.at[pl.ds(NS * RPT, RREM)],
                        agg.at[pl.ds(NS * RPT, RREM)])

    pltpu.sync_copy(dst3.at[wid], di)
    plsc.subcore_barrier()

    ebase = wid * EPW
    pltpu.async_copy(m_hbm.at[pl.ds(ebase, CH)], mva, semma)

    def body(j2, carry):
        ja = 2 * j2
        jb = ja + 1
        pltpu.async_copy(m_hbm.at[pl.ds(ebase + jb * CH, CH)], mvb, semmb)
        pltpu.make_async_copy(m_hbm.at[pl.ds(0, CH)], mva, semma).wait()
        pltpu.sync_copy(mva, agg.at[di.at[ja]], add=True)
        pltpu.async_copy(m_hbm.at[pl.ds(ebase + (ja + 2) * CH, CH)], mva, semma)
        pltpu.make_async_copy(m_hbm.at[pl.ds(0, CH)], mvb, semmb).wait()
        pltpu.sync_copy(mvb, agg.at[di.at[jb]], add=True)
        return carry

    lax.fori_loop(0, (NCH - 1) // 2, body, 0)
    pltpu.make_async_copy(m_hbm.at[pl.ds(0, CH)], mva, semma).wait()
    pltpu.sync_copy(mva, agg.at[di.at[NCH - 1]], add=True)
    plsc.subcore_barrier()
    pltpu.sync_copy(agg.at[pl.ds(s * RPT, RPT)], out.at[c].at[pl.ds(s * RPT, RPT)])

    @pl.when(s == 0)
    def _():
        pltpu.sync_copy(agg.at[pl.ds(NS * RPT, RREM)],
                        out.at[c].at[pl.ds(NS * RPT, RREM)])


def _scatter_call(m, dst3, zeros_n):
    fn = pl.kernel(
        _scatter_body,
        out_type=jax.ShapeDtypeStruct((NC, N, H), jnp.float32),
        mesh=_mesh(),
        compiler_params=pltpu.CompilerParams(needs_layout_passes=False),
        scratch_types=[
            pltpu.VMEM((NCH, CH), jnp.int32),
            pltpu.VMEM((CH, H), jnp.float32),
            pltpu.VMEM((CH, H), jnp.float32),
            pltpu.VMEM_SHARED((N, H), jnp.float32),
            pltpu.SemaphoreType.DMA,
            pltpu.SemaphoreType.DMA,
        ],
    )
    return fn(m, dst3, zeros_n)


def _embed_body(a_ref, emb_ref, o_ref):
    oh = (a_ref[...] == lax.broadcasted_iota(jnp.int32, (1, 128), 1)).astype(jnp.float32)
    o_ref[...] = jnp.dot(oh, emb_ref[...], preferred_element_type=jnp.float32, precision=lax.Precision.HIGHEST)


def _embed_call(atoms2, embp):
    return pl.pallas_call(
        _embed_body,
        grid=(N // BN,),
        in_specs=[
            pl.BlockSpec((BN, 1), lambda i: (i, 0)),
            pl.BlockSpec((128, H), lambda i: (0, 0)),
        ],
        out_specs=pl.BlockSpec((BN, H), lambda i: (i, 0)),
        out_shape=jax.ShapeDtypeStruct((N, H), jnp.float32),
    )(atoms2, embp)


def _proj_body(x_ref, p0_ref, p1_ref, wd_ref, bd_ref, ws_ref, xn_ref, d_ref, s_ref):
    xn = x_ref[...] + p0_ref[...] + p1_ref[...]
    xn_ref[...] = xn
    d_ref[...] = jnp.dot(xn, wd_ref[...], preferred_element_type=jnp.float32) + bd_ref[...]
    s_ref[...] = jnp.dot(xn, ws_ref[...], preferred_element_type=jnp.float32)


def _proj_call(x, p0, p1, wd, bd, ws):
    return pl.pallas_call(
        _proj_body,
        grid=(N // BN,),
        in_specs=[
            pl.BlockSpec((BN, H), lambda i: (i, 0)),
            pl.BlockSpec((BN, H), lambda i: (i, 0)),
            pl.BlockSpec((BN, H), lambda i: (i, 0)),
            pl.BlockSpec((H, 2 * H), lambda i: (0, 0)),
            pl.BlockSpec((1, 2 * H), lambda i: (0, 0)),
            pl.BlockSpec((H, 2 * H), lambda i: (0, 0)),
        ],
        out_specs=[
            pl.BlockSpec((BN, H), lambda i: (i, 0)),
            pl.BlockSpec((BN, 2 * H), lambda i: (i, 0)),
            pl.BlockSpec((BN, 2 * H), lambda i: (i, 0)),
        ],
        out_shape=[
            jax.ShapeDtypeStruct((N, H), jnp.float32),
            jax.ShapeDtypeStruct((N, 2 * H), jnp.float32),
            jax.ShapeDtypeStruct((N, 2 * H), jnp.float32),
        ],
    )(x, p0, p1, wd, bd, ws)


def _act_body(c_ref, dx_ref, dy_ref, dz_ref, wf2_ref, ws2_ref, m_ref):
    cb = c_ref[...]
    dx = dx_ref[...]
    dy = dy_ref[...]
    dz = dz_ref[...]
    e = jnp.sqrt(dx * dx + dy * dy + dz * dz)
    u = cb[:, :H] + e * wf2_ref[...]
    v = cb[:, H:] + e * ws2_ref[...]
    gate = 1.0 / (1.0 + jnp.exp(-u))
    core = jnp.maximum(v, 0.0) + jnp.log(1.0 + jnp.exp(-jnp.abs(v)))
    m_ref[...] = gate * core


def _act_call(c_arr, dxc, dyc, dzc, wf2, ws2):
    return pl.pallas_call(
        _act_body,
        grid=(E // BE,),
        in_specs=[
            pl.BlockSpec((BE, 2 * H), lambda i: (i, 0)),
            pl.BlockSpec((BE, 1), lambda i: (i, 0)),
            pl.BlockSpec((BE, 1), lambda i: (i, 0)),
            pl.BlockSpec((BE, 1), lambda i: (i, 0)),
            pl.BlockSpec((1, H), lambda i: (0, 0)),
            pl.BlockSpec((1, H), lambda i: (0, 0)),
        ],
        out_specs=pl.BlockSpec((BE, H), lambda i: (i, 0)),
        out_shape=jax.ShapeDtypeStruct((E, H), jnp.float32),
    )(c_arr, dxc, dyc, dzc, wf2, ws2)


def _pool_body(x_ref, p0_ref, p1_ref, b_ref, fcw_ref, fcb_ref, ow_ref, ob_ref,
               o_ref, acc, cnt):
    i = pl.program_id(0)

    @pl.when(i == 0)
    def _():
        acc[...] = jnp.zeros_like(acc)
        cnt[...] = jnp.zeros_like(cnt)

    xb = x_ref[...] + p0_ref[...] + p1_ref[...]
    oh = (b_ref[...] == lax.broadcasted_iota(jnp.int32, (1, G), 1)).astype(jnp.float32)
    acc[...] += lax.dot_general(oh, xb, (((0,), (0,)), ((), ())),
                                preferred_element_type=jnp.float32, precision=lax.Precision.HIGHEST)
    cnt[...] += lax.dot_general(oh, jnp.ones((BN, 1), jnp.float32),
                                (((0,), (0,)), ((), ())),
                                preferred_element_type=jnp.float32, precision=lax.Precision.HIGHEST)

    @pl.when(i == pl.num_programs(0) - 1)
    def _():
        h = acc[...] / jnp.maximum(cnt[...], 1.0)
        for l in range(F):
            h = jnp.dot(h, fcw_ref[l], preferred_element_type=jnp.float32) + fcb_ref[l]
        o_ref[...] = jnp.dot(h, ow_ref[...], preferred_element_type=jnp.float32) + ob_ref[...]


def _pool_call(x, p0, p1, batch2, fcW, fcb3, outW, outb2):
    return pl.pallas_call(
        _pool_body,
        grid=(N // BN,),
        in_specs=[
            pl.BlockSpec((BN, H), lambda i: (i, 0)),
            pl.BlockSpec((BN, H), lambda i: (i, 0)),
            pl.BlockSpec((BN, H), lambda i: (i, 0)),
            pl.BlockSpec((BN, 1), lambda i: (i, 0)),
            pl.BlockSpec((F, H, H), lambda i: (0, 0, 0)),
            pl.BlockSpec((F, 1, H), lambda i: (0, 0, 0)),
            pl.BlockSpec((H, 1), lambda i: (0, 0)),
            pl.BlockSpec((1, 1), lambda i: (0, 0)),
        ],
        out_specs=pl.BlockSpec((G, 1), lambda i: (0, 0)),
        out_shape=jax.ShapeDtypeStruct((G, 1), jnp.float32),
        scratch_shapes=[
            pltpu.VMEM((G, H), jnp.float32),
            pltpu.VMEM((G, 1), jnp.float32),
        ],
    )(x, p0, p1, batch2, fcW, fcb3, outW, outb2)


def kernel(atoms, pos, edge_index, batch, emb, Wf, bf, Ws, bs, fcW, fcb, outW, outb):
    atoms2 = atoms.astype(jnp.int32).reshape(N, 1)
    src = edge_index[0].astype(jnp.int32)
    dst = edge_index[1].astype(jnp.int32)
    batch2 = batch.astype(jnp.int32).reshape(N, 1)
    posx = pos[:, 0]
    posy = pos[:, 1]
    posz = pos[:, 2]
    embp = jnp.pad(emb, ((0, 128 - emb.shape[0]), (0, 0)))

    WD = jnp.concatenate([Wf[:, :H, :], Ws[:, :H, :]], axis=2)
    WSrc = jnp.concatenate([Wf[:, H:2 * H, :], Ws[:, H:2 * H, :]], axis=2)
    bD = jnp.concatenate([bf, bs], axis=1).reshape(L, 1, 2 * H)
    wf2 = Wf[:, 2 * H, :].reshape(L, 1, H)
    ws2 = Ws[:, 2 * H, :].reshape(L, 1, H)

    dst3 = dst.reshape(NW, NCH, CH)
    src3 = src.reshape(NW, NCH, CH)
    zeros_n = jnp.zeros((N, H), jnp.float32)

    dxe, dye, dze = _diff_call(posx, posy, posz, src, dst)
    dxc = dxe.reshape(E, 1)
    dyc = dye.reshape(E, 1)
    dzc = dze.reshape(E, 1)

    x = _embed_call(atoms2, embp)
    p0 = zeros_n
    p1 = zeros_n
    for l in range(L):
        x, d_tab, s_tab = _proj_call(x, p0, p1, WD[l], bD[l], WSrc[l])
        c_arr = _gather_call(d_tab, s_tab, dst3, src3)
        m = _act_call(c_arr, dxc, dyc, dzc, wf2[l], ws2[l])
        parts = _scatter_call(m, dst3, zeros_n)
        p0 = parts[0]
        p1 = parts[1]

    return _pool_call(x, p0, p1, batch2, fcW, fcb.reshape(F, 1, H), outW,
                      outb.reshape(1, 1))
